# Initial kernel scaffold; baseline (speedup 1.0000x reference)
import jax
import jax.numpy as jnp
from jax.experimental import pallas as pl


def kernel(mu, sigma, weight, sigma_w):
    raise NotImplementedError("write your pallas kernel here")


# speedup vs baseline: 1.1140x; 1.1140x over previous
import jax
import jax.numpy as jnp
from jax import lax
from jax.experimental import pallas as pl
from jax.experimental.pallas import tpu as pltpu


def _im2col(x, k):
    B, C, H, W = x.shape
    Ho, Wo = H - k + 1, W - k + 1
    cols = []
    for kh in range(k):
        for kw in range(k):
            cols.append(x[:, :, kh:kh + Ho, kw:kw + Wo])
    patches = jnp.stack(cols, axis=2)
    return patches.reshape(B, C * k * k, Ho * Wo)


def _row_tile(L, max_tile=128):
    best = None
    for tl in range(8, min(L, max_tile) + 1, 8):
        if L % tl == 0:
            best = tl
    return best if best is not None else L


def _evi_kernel(pcol_ref, p_ref, sp_ref, w_ref, w2_ref, splus_ref,
                mu_z_ref, sigma_z_ref, diag_ref):
    f32 = jnp.float32
    t = pl.program_id(1)
    TL = sigma_z_ref.shape[2]
    L = sigma_z_ref.shape[3]

    @pl.when(t == 0)
    def _():
        p = p_ref[0]
        sp = sp_ref[0]
        mu_z = jnp.dot(w_ref[...], p, preferred_element_type=f32)
        mu_z_ref[...] = mu_z[None]
        mw = jnp.dot(w2_ref[...], sp, preferred_element_type=f32)
        tr = jnp.sum(sp, axis=0, keepdims=True)
        diag_ref[...] = mw + splus_ref[...] * tr

    xm = lax.dot_general(pcol_ref[0], p_ref[0],
                         (((0,), (0,)), ((), ())),
                         preferred_element_type=f32)

    row_g = t * TL + lax.broadcasted_iota(jnp.int32, (TL, L), 0)
    col = lax.broadcasted_iota(jnp.int32, (TL, L), 1)
    on_diag = row_g == col

    splus = splus_ref[...]
    dvals = diag_ref[...]
    out = (splus[:, :, None] * xm[None]
           + jnp.where(on_diag[None], dvals[:, None, :], jnp.zeros((), f32)))
    sigma_z_ref[0] = out.astype(sigma_z_ref.dtype)


def kernel(mu, sigma, weight, sigma_w):
    f32 = jnp.float32
    B, C_in, H, W = mu.shape
    C_out = weight.shape[0]
    k = weight.shape[2]
    Ho, Wo = H - k + 1, W - k + 1
    L = Ho * Wo
    K = C_in * k * k
    TL = _row_tile(L)
    T = L // TL

    patches = _im2col(mu.astype(f32), k)
    diag_sigma = jnp.diagonal(sigma, axis1=2, axis2=3)
    diag_sigma = diag_sigma.reshape(B, C_in, H, W)
    sig_patches = _im2col(diag_sigma.astype(f32), k)

    w_flat = weight.reshape(C_out, K).astype(f32)
    w2_flat = w_flat * w_flat
    splus = jax.nn.softplus(sigma_w.astype(f32)).reshape(C_out, 1)

    cost = pl.CostEstimate(
        flops=int(B * (2 * L * K * L + 4 * C_out * K * L + 3 * C_out * L * L)),
        transcendentals=0,
        bytes_accessed=int(B * (3 * K * L + C_out * L * L + C_out * L) * 4),
    )

    mu_z_flat, sigma_z = pl.pallas_call(
        _evi_kernel,
        out_shape=(jax.ShapeDtypeStruct((B, C_out, L), f32),
                   jax.ShapeDtypeStruct((B, C_out, L, L), f32)),
        grid_spec=pltpu.PrefetchScalarGridSpec(
            num_scalar_prefetch=0,
            grid=(B, T),
            in_specs=[
                pl.BlockSpec((1, K, TL), lambda b, t: (b, 0, t)),
                pl.BlockSpec((1, K, L), lambda b, t: (b, 0, 0)),
                pl.BlockSpec((1, K, L), lambda b, t: (b, 0, 0)),
                pl.BlockSpec((C_out, K), lambda b, t: (0, 0)),
                pl.BlockSpec((C_out, K), lambda b, t: (0, 0)),
                pl.BlockSpec((C_out, 1), lambda b, t: (0, 0)),
            ],
            out_specs=[
                pl.BlockSpec((1, C_out, L), lambda b, t: (b, 0, 0)),
                pl.BlockSpec((1, C_out, TL, L), lambda b, t: (b, 0, t, 0)),
            ],
            scratch_shapes=[pltpu.VMEM((C_out, L), f32)],
        ),
        compiler_params=pltpu.CompilerParams(
            dimension_semantics=("parallel", "arbitrary")),
        cost_estimate=cost,
    )(patches, patches, sig_patches, w_flat, w2_flat, splus)

    mu_z = mu_z_flat.reshape(B, C_out, Ho, Wo)
    return mu_z, sigma_z
